# Initial kernel scaffold; baseline (speedup 1.0000x reference)
import jax
import jax.numpy as jnp
from jax.experimental import pallas as pl


def kernel(X, v_idx, e_idx, e_weight, W1e, b1e, g1e, bb1e, W2e, b2e, g2e, bb2e, W_fc, b_fc, W_fc1u, b_fc1u, W_fc2u, b_fc2u, W_fc1r, b_fc1r, W_fc2r, b_fc2r, W_fc1, b_fc1, W_fc2, b_fc2, W1d, b1d, g1d, bb1d, W2d, b2d, g2d, bb2d):
    raise NotImplementedError("write your pallas kernel here")


# speedup vs baseline: 4.7511x; 4.7511x over previous
import functools

import jax
import jax.numpy as jnp
from jax import lax
from jax.experimental import pallas as pl
from jax.experimental.pallas import tpu as pltpu
from jax.experimental.pallas import tpu_sc as plsc

NV = 10000
NE = 5000
NNZ = 320000
NC = 2
NT = 16
L = 16
W = 128

CHUNK = 128
NCHUNK = NNZ // CHUNK
NEP = 5120
EPT = NEP // NT
NVP = 10240
VPT = NVP // NT
WCH = 80


def _mesh():
    return plsc.VectorSubcoreMesh(
        core_axis_name="c", subcore_axis_name="s", num_cores=NC, num_subcores=NT)


def _deg_body(vidx_hbm, eidx_hbm, we_hbm, se_out, rv_out,
              edeg, vdeg, vi, ei, ones, dbuf, obuf, wbuf):
    c = lax.axis_index("c")
    t = lax.axis_index("s")
    for j in range(CHUNK // L):
        ones[pl.ds(j * L, L)] = jnp.ones((L,), jnp.float32)
    for j in range(VPT // L):
        dbuf[pl.ds(j * L, L)] = jnp.zeros((L,), jnp.float32)
    pltpu.sync_copy(dbuf.at[pl.ds(0, EPT)], edeg.at[pl.ds(t * EPT, EPT)])
    pltpu.sync_copy(dbuf, vdeg.at[pl.ds(t * VPT, VPT)])
    plsc.subcore_barrier()

    def body(k, carry):
        g = k * NT + t

        @pl.when(g < NCHUNK)
        def _():
            base = g * CHUNK
            pltpu.sync_copy(vidx_hbm.at[pl.ds(base, CHUNK)], vi)
            pltpu.sync_copy(eidx_hbm.at[pl.ds(base, CHUNK)], ei)
            pltpu.sync_copy(ones, edeg.at[ei], add=True)
            pltpu.sync_copy(ones, vdeg.at[vi], add=True)
        return carry

    lax.fori_loop(0, -(-NCHUNK // NT), body, 0)
    plsc.subcore_barrier()

    @pl.when(c == 0)
    def _():
        pltpu.sync_copy(edeg.at[pl.ds(t * EPT, EPT)], dbuf.at[pl.ds(0, EPT)])
        pltpu.sync_copy(we_hbm.at[pl.ds(t * EPT, EPT)], wbuf)
        for j in range(EPT // L):
            d = dbuf[pl.ds(j * L, L)]
            w = wbuf[pl.ds(j * L, L)]
            obuf[pl.ds(j * L, L)] = w / jnp.maximum(d, 1.0)
        pltpu.sync_copy(obuf.at[pl.ds(0, EPT)], se_out.at[pl.ds(t * EPT, EPT)])

    @pl.when(c == 1)
    def _():
        pltpu.sync_copy(vdeg.at[pl.ds(t * VPT, VPT)], dbuf)
        for j in range(VPT // L):
            d = dbuf[pl.ds(j * L, L)]
            obuf[pl.ds(j * L, L)] = 1.0 / jnp.maximum(d, 1.0)
        pltpu.sync_copy(obuf, rv_out.at[pl.ds(t * VPT, VPT)])


@functools.cache
def _get_deg_kernel():
    return pl.kernel(
        _deg_body,
        out_type=(jax.ShapeDtypeStruct((NEP,), jnp.float32),
                  jax.ShapeDtypeStruct((NVP,), jnp.float32)),
        mesh=_mesh(),
        scratch_types=[
            pltpu.VMEM_SHARED((NEP,), jnp.float32),
            pltpu.VMEM_SHARED((NVP,), jnp.float32),
            pltpu.VMEM((CHUNK,), jnp.int32),
            pltpu.VMEM((CHUNK,), jnp.int32),
            pltpu.VMEM((CHUNK,), jnp.float32),
            pltpu.VMEM((VPT,), jnp.float32),
            pltpu.VMEM((VPT,), jnp.float32),
            pltpu.VMEM((EPT,), jnp.float32),
        ],
        name="hg_degree_scales",
    )


def _seg_body(cfg, table_hbm, gidx_hbm, sidx_hbm, out_hbm,
              acc, gbuf, sbuf, rows, zbuf):
    split_cols, trh, nr, wn = cfg
    c = lax.axis_index("c")
    t = lax.axis_index("s")
    pcc = NCHUNK if split_cols else NCHUNK // NC
    start = 0 if split_cols else c * pcc
    goff = c * trh if split_cols else 0
    rpt = nr // NT

    for r in range(L):
        for j in range(W // L):
            zbuf[r, pl.ds(j * L, L)] = jnp.zeros((L,), jnp.float32)
    for k in range(rpt // L):
        pltpu.sync_copy(zbuf, acc.at[pl.ds(t * rpt + k * L, L)])
    plsc.subcore_barrier()

    def step(k, carry):
        rel = k * NT + t

        @pl.when(rel < pcc)
        def _():
            base = (start + rel) * CHUNK
            pltpu.sync_copy(gidx_hbm.at[pl.ds(base, CHUNK)], gbuf)
            pltpu.sync_copy(sidx_hbm.at[pl.ds(base, CHUNK)], sbuf)

            def off(j, cc):
                gbuf[pl.ds(j * L, L)] = gbuf[pl.ds(j * L, L)] + goff
                return cc

            if split_cols:
                lax.fori_loop(0, CHUNK // L, off, 0)
            pltpu.sync_copy(table_hbm.at[gbuf], rows)
            pltpu.sync_copy(rows, acc.at[sbuf], add=True)
        return carry

    lax.fori_loop(0, -(-pcc // NT), step, 0)
    plsc.subcore_barrier()

    def wout(k, carry):
        g = k * NT + t

        @pl.when(g < wn // WCH)
        def _():
            pltpu.sync_copy(acc.at[pl.ds(g * WCH, WCH)],
                            out_hbm.at[pl.ds(c * wn + g * WCH, WCH)])
        return carry

    lax.fori_loop(0, -(-(wn // WCH) // NT), wout, 0)


@functools.cache
def _make_seg(split_cols, trh, nr, wn):
    return pl.kernel(
        functools.partial(_seg_body, (split_cols, trh, nr, wn)),
        out_type=jax.ShapeDtypeStruct((NC * wn, W), jnp.float32),
        mesh=_mesh(),
        scratch_types=[
            pltpu.VMEM_SHARED((nr, W), jnp.float32),
            pltpu.VMEM((CHUNK,), jnp.int32),
            pltpu.VMEM((CHUNK,), jnp.int32),
            pltpu.VMEM((CHUNK, W), jnp.float32),
            pltpu.VMEM((L, W), jnp.float32),
        ],
        name=f"hg_seg_{'cs' if split_cols else 'ps'}_{trh}_{nr}",
    )


def _dot(a, b):
    return jnp.dot(a, b, preferred_element_type=jnp.float32,
                   precision=lax.Precision.HIGHEST)


def _bn(y, g, b):
    m = jnp.mean(y, axis=0)
    v = jnp.mean((y - m[None, :]) ** 2, axis=0)
    return (y - m[None, :]) * lax.rsqrt(v[None, :] + 1e-5) * g[None, :] + b[None, :]


def _halves(y):
    d = y.shape[1] // 2
    return y[:, :d], y[:, d:]


def _merge_cs(v_ref, rv_ref):
    return jnp.concatenate([v_ref[0], v_ref[1]], axis=1) * rv_ref[...][:, None]


def _merge_ps(v_ref, rv_ref, d):
    return (v_ref[0] + v_ref[1])[:, :d] * rv_ref[...][:, None]


def _tc1_body(x_ref, w_ref, b_ref, g_ref, bb_ref, o_ref):
    y = _bn(_dot(x_ref[...], w_ref[...]) + b_ref[...][None, :],
            g_ref[...], bb_ref[...])
    lo, hi = _halves(y)
    o_ref[0] = lo
    o_ref[1] = hi


def _tc2_body(v_ref, rv_ref, w2_ref, b2_ref, g2_ref, bb2_ref,
              wfc_ref, bfc_ref, z2_ref, e1_ref):
    rv = rv_ref[...][:, None]
    x1lo = jnp.maximum(v_ref[0] * rv, 0.0)
    x1hi = jnp.maximum(v_ref[1] * rv, 0.0)
    w2 = w2_ref[...]
    y = _bn(_dot(x1lo, w2[:128]) + _dot(x1hi, w2[128:]) + b2_ref[...][None, :],
            g2_ref[...], bb2_ref[...])
    z2_ref[:, :64] = y
    z2_ref[:, 64:] = jnp.zeros_like(y)
    wfc = wfc_ref[...]
    e1_ref[...] = (_dot(x1lo, wfc[:128]) + _dot(x1hi, wfc[128:])
                   + bfc_ref[...][None, :])


def _tc3a_body(v_ref, rv_ref, e1_ref,
               wf1u, bf1u, wf2u, bf2u, wf1r, bf1r, wf2r, bf2r,
               wf1, bf1, wf2, bf2, x_ref):
    xe = (v_ref[0] + v_ref[1])[:, :64] * rv_ref[...]
    e1 = e1_ref[...]

    def aff(a, w, b):
        return _dot(a, w[...]) + b[...][None, :]

    z = jax.nn.sigmoid(aff(e1, wf1u, bf1u) + aff(xe, wf2u, bf2u))
    r = jax.nn.sigmoid(aff(e1, wf1r, bf1r) + aff(xe, wf2r, bf2r))
    h = jnp.tanh(aff(e1, wf1, bf1) + aff(r * xe, wf2, bf2))
    x_ref[...] = (1.0 - z) * xe + z * h


def _tc3b_body(x_ref, w1d, b1d, g1d, bb1d, z3_ref):
    y = _bn(_dot(x_ref[...], w1d[...]) + b1d[...][None, :],
            g1d[...], bb1d[...])
    lo, hi = _halves(y)
    z3_ref[0] = lo
    z3_ref[1] = hi


def _tc4_body(v_ref, rv_ref, w_ref, b_ref, g_ref, bb_ref, o_ref):
    x3 = jnp.maximum(_merge_cs(v_ref, rv_ref), 0.0)
    o_ref[...] = _bn(_dot(x3, w_ref[...]) + b_ref[...][None, :],
                     g_ref[...], bb_ref[...])


def _tc5_body(v_ref, rv_ref, o_ref):
    o_ref[...] = _merge_ps(v_ref, rv_ref, 128)


def _scale_cs_body(e_ref, se_ref, o_ref):
    o_ref[...] = e_ref[...] * se_ref[...][None, :, None]


def _scale_ps_body(e_ref, se_ref, o_ref):
    o_ref[...] = (e_ref[0] + e_ref[1]) * se_ref[...][:, None]


def _sds(*shape):
    return jax.ShapeDtypeStruct(shape, jnp.float32)


def _v2v_cs(z_split, v_idx, e_idx, se_pad):
    eraw = _make_seg(True, NV, NEP, NEP)(
        z_split.reshape(NC * NV, W), v_idx, e_idx)
    y = pl.pallas_call(_scale_cs_body, out_shape=_sds(2, NEP, W))(
        eraw.reshape(2, NEP, W), se_pad)
    return _make_seg(True, NEP, NVP, NV)(
        y.reshape(NC * NEP, W), e_idx, v_idx).reshape(2, NV, W)


def _v2v_ps(z, v_idx, e_idx, se_pad):
    eraw = _make_seg(False, NV, NEP, NEP)(z, v_idx, e_idx)
    y = pl.pallas_call(_scale_ps_body, out_shape=_sds(NEP, W))(
        eraw.reshape(2, NEP, W), se_pad)
    return _make_seg(False, NEP, NVP, NV)(y, e_idx, v_idx).reshape(2, NV, W)


def kernel(X, v_idx, e_idx, e_weight, W1e, b1e, g1e, bb1e, W2e, b2e, g2e, bb2e,
           W_fc, b_fc, W_fc1u, b_fc1u, W_fc2u, b_fc2u, W_fc1r, b_fc1r,
           W_fc2r, b_fc2r, W_fc1, b_fc1, W_fc2, b_fc2, W1d, b1d, g1d, bb1d,
           W2d, b2d, g2d, bb2d):
    we_pad = jnp.pad(e_weight, (0, NEP - NE))
    se_pad, rv_pad = _get_deg_kernel()(v_idx, e_idx, we_pad)
    rv = rv_pad[:NV]

    z1 = pl.pallas_call(_tc1_body, out_shape=_sds(2, NV, 128))(
        X, W1e, b1e, g1e, bb1e)
    v1 = _v2v_cs(z1, v_idx, e_idx, se_pad)

    z2, e1 = pl.pallas_call(
        _tc2_body, out_shape=(_sds(NV, 128), _sds(NV, 64)))(
        v1, rv, W2e, b2e, g2e, bb2e, W_fc, b_fc)
    v2 = _v2v_ps(z2, v_idx, e_idx, se_pad)

    BR = 2000
    wspec = [pl.BlockSpec(w.shape, lambda i, n=w.ndim: (0,) * n)
             for w in (W_fc1u, b_fc1u, W_fc2u, b_fc2u, W_fc1r, b_fc1r,
                       W_fc2r, b_fc2r, W_fc1, b_fc1, W_fc2, b_fc2)]
    xfused = pl.pallas_call(
        _tc3a_body,
        grid=(NV // BR,),
        in_specs=[pl.BlockSpec((2, BR, 128), lambda i: (0, i, 0)),
                  pl.BlockSpec((BR, 1), lambda i: (i, 0)),
                  pl.BlockSpec((BR, 64), lambda i: (i, 0))] + wspec,
        out_specs=pl.BlockSpec((BR, 64), lambda i: (i, 0)),
        out_shape=_sds(NV, 64))(
        v2, rv[:, None], e1,
        W_fc1u, b_fc1u, W_fc2u, b_fc2u, W_fc1r, b_fc1r, W_fc2r, b_fc2r,
        W_fc1, b_fc1, W_fc2, b_fc2)
    z3 = pl.pallas_call(_tc3b_body, out_shape=_sds(2, NV, 128))(
        xfused, W1d, b1d, g1d, bb1d)
    v3 = _v2v_cs(z3, v_idx, e_idx, se_pad)

    z4 = pl.pallas_call(_tc4_body, out_shape=_sds(NV, 128))(
        v3, rv, W2d, b2d, g2d, bb2d)
    v4 = _v2v_ps(z4, v_idx, e_idx, se_pad)

    out = pl.pallas_call(_tc5_body, out_shape=_sds(NV, 128))(v4, rv)
    return out
